# baseline (speedup 1.0000x reference)
import functools
import math

import jax
import jax.numpy as jnp
from jax import lax
from jax.experimental import pallas as pl
from jax.experimental.pallas import tpu as pltpu
from jax.experimental.pallas import tpu_sc as plsc

_N = 32768
_K = math.ceil(0.1 * _N)
_ROWS = 64
_NTILES = 32
_RPT = _ROWS // _NTILES
_CHUNKS = _N // 16

_mesh = plsc.VectorSubcoreMesh(core_axis_name="c", subcore_axis_name="s")


def _sc_body(x_hbm, out_hbm, data_v, res_v):
    wid = lax.axis_index("s") * 2 + lax.axis_index("c")
    base = wid * _RPT
    pltpu.sync_copy(x_hbm.at[pl.ds(base, _RPT)], data_v)

    res = jnp.zeros((16,), jnp.float32)
    lane = lax.iota(jnp.int32, 16)

    for r in range(_RPT):
        def relu_step(i, tot):
            v = jnp.maximum(data_v[r, pl.ds(i * 16, 16)], 0.0)
            data_v[r, pl.ds(i * 16, 16)] = v
            return tot + v

        tot_vec = lax.fori_loop(0, _CHUNKS, relu_step, jnp.zeros((16,), jnp.float32))
        tot = jnp.sum(tot_vec)

        def bs_step(_, carry):
            lo, hi = carry
            mid = lo + ((hi - lo) >> 1)

            def cnt_step(i, cnt):
                b = plsc.bitcast(data_v[r, pl.ds(i * 16, 16)], jnp.int32)
                return cnt + jnp.where(b >= mid, 1, 0)

            cnt = lax.fori_loop(0, _CHUNKS, cnt_step, jnp.zeros((16,), jnp.int32))
            ge = jnp.sum(cnt) >= _K
            return jnp.where(ge, mid, lo), jnp.where(ge, hi, mid)

        lo, _hi = lax.fori_loop(
            0, 31, bs_step, (jnp.int32(0), jnp.int32(0x7F800000))
        )
        t_vec = plsc.bitcast(jnp.full((16,), lo, jnp.int32), jnp.float32)

        def fin_step(i, carry):
            s, c = carry
            v = data_v[r, pl.ds(i * 16, 16)]
            gt = plsc.bitcast(v, jnp.int32) > lo
            return s + jnp.where(gt, v, 0.0), c + jnp.where(gt, 1.0, 0.0)

        s_vec, c_vec = lax.fori_loop(
            0, _CHUNKS, fin_step,
            (jnp.zeros((16,), jnp.float32), jnp.zeros((16,), jnp.float32)),
        )
        s = jnp.sum(s_vec)
        c = jnp.sum(c_vec)
        t = t_vec[0]
        topk = s + t * (_K - c)
        cost = topk * (1.0 / _K) - (tot - topk) * (1.0 / (_N - _K))
        res = res + jnp.where(lane == r, cost, 0.0)

    res_v[...] = res
    pltpu.sync_copy(res_v, out_hbm.at[wid])


_sc_call = functools.partial(
    pl.kernel,
    out_type=jax.ShapeDtypeStruct((_NTILES, 16), jnp.float32),
    mesh=_mesh,
    compiler_params=pltpu.CompilerParams(needs_layout_passes=False),
    scratch_types=[
        pltpu.VMEM((_RPT, _N), jnp.float32),
        pltpu.VMEM((16,), jnp.float32),
    ],
)


def _tc_mean_body(p_ref, o_ref):
    o_ref[...] = (jnp.sum(p_ref[...]) / _ROWS).reshape(1, 1)


def kernel(input):
    partials = _sc_call(_sc_body)(input)
    out = pl.pallas_call(
        _tc_mean_body,
        out_shape=jax.ShapeDtypeStruct((1, 1), jnp.float32),
    )(partials)
    return out[0, 0]


# speedup vs baseline: 4.5767x; 4.5767x over previous
import functools
import math

import jax
import jax.numpy as jnp
from jax import lax
from jax.experimental import pallas as pl
from jax.experimental.pallas import tpu as pltpu
from jax.experimental.pallas import tpu_sc as plsc

_N = 32768
_K = math.ceil(0.1 * _N)
_ROWS = 64
_NTILES = 32
_RPT = _ROWS // _NTILES
_CHUNKS = _N // 16

_mesh = plsc.VectorSubcoreMesh(core_axis_name="c", subcore_axis_name="s")


def _sc_body(x_hbm, out_hbm, data_v, res_v):
    wid = lax.axis_index("s") * 2 + lax.axis_index("c")
    base = wid * _RPT
    pltpu.sync_copy(x_hbm.at[pl.ds(base, _RPT)], data_v)

    res = jnp.zeros((16,), jnp.float32)
    lane = lax.iota(jnp.int32, 16)

    for r in range(_RPT):
        @plsc.parallel_loop(0, _N, step=16, unroll=8,
                            carry=jnp.zeros((16,), jnp.float32))
        def tot_vec(i, tot):
            v = jnp.maximum(data_v[r, pl.ds(i, 16)], 0.0)
            data_v[r, pl.ds(i, 16)] = v
            return tot + v

        tot = jnp.sum(tot_vec)

        def bs_step(_, carry):
            lo, hi = carry
            mid = lo + ((hi - lo) >> 1)

            @plsc.parallel_loop(0, _N, step=16, unroll=8,
                                carry=jnp.zeros((16,), jnp.int32))
            def cnt(i, acc):
                b = plsc.bitcast(data_v[r, pl.ds(i, 16)], jnp.int32)
                return acc + jnp.where(b >= mid, 1, 0)

            ge = jnp.sum(cnt) >= _K
            return jnp.where(ge, mid, lo), jnp.where(ge, hi, mid)

        lo, _hi = lax.fori_loop(
            0, 31, bs_step, (jnp.int32(0), jnp.int32(0x7F800000))
        )
        t_vec = plsc.bitcast(jnp.full((16,), lo, jnp.int32), jnp.float32)

        @plsc.parallel_loop(0, _N, step=16, unroll=8,
                            carry=(jnp.zeros((16,), jnp.float32),
                                   jnp.zeros((16,), jnp.float32)))
        def sc_pair(i, carry):
            s, c = carry
            v = data_v[r, pl.ds(i, 16)]
            gt = plsc.bitcast(v, jnp.int32) > lo
            return s + jnp.where(gt, v, 0.0), c + jnp.where(gt, 1.0, 0.0)

        s_vec, c_vec = sc_pair
        s = jnp.sum(s_vec)
        c = jnp.sum(c_vec)
        t = t_vec[0]
        topk = s + t * (_K - c)
        cost = topk * (1.0 / _K) - (tot - topk) * (1.0 / (_N - _K))
        res = res + jnp.where(lane == r, cost, 0.0)

    res_v[...] = res
    pltpu.sync_copy(res_v, out_hbm.at[wid])


_sc_call = functools.partial(
    pl.kernel,
    out_type=jax.ShapeDtypeStruct((_NTILES, 16), jnp.float32),
    mesh=_mesh,
    compiler_params=pltpu.CompilerParams(needs_layout_passes=False),
    scratch_types=[
        pltpu.VMEM((_RPT, _N), jnp.float32),
        pltpu.VMEM((16,), jnp.float32),
    ],
)


def _tc_mean_body(p_ref, o_ref):
    o_ref[...] = (jnp.sum(p_ref[...]) / _ROWS).reshape(1, 1)


def kernel(input):
    partials = _sc_call(_sc_body)(input)
    out = pl.pallas_call(
        _tc_mean_body,
        out_shape=jax.ShapeDtypeStruct((1, 1), jnp.float32),
    )(partials)
    return out[0, 0]
